# baseline (speedup 1.0000x reference)
import jax
import jax.numpy as jnp
from jax.experimental import pallas as pl


S_BLK = 1024


def _pe_add_kernel(scale_ref, in_ref, pe_ref, out_ref):
    s = scale_ref[0]
    out_ref[...] = in_ref[...] + s * pe_ref[...][None, :, :]


def kernel(input, pe, scale_param):
    batch, seq, dim = input.shape
    grid = (seq // S_BLK,)
    return pl.pallas_call(
        _pe_add_kernel,
        grid=grid,
        in_specs=[
            pl.BlockSpec((1,), lambda i: (0,)),
            pl.BlockSpec((batch, S_BLK, dim), lambda i: (0, i, 0)),
            pl.BlockSpec((S_BLK, dim), lambda i: (i, 0)),
        ],
        out_specs=pl.BlockSpec((batch, S_BLK, dim), lambda i: (0, i, 0)),
        out_shape=jax.ShapeDtypeStruct((batch, seq, dim), input.dtype),
    )(scale_param, input, pe[:seq])


# speedup vs baseline: 1.0110x; 1.0110x over previous
import jax
import jax.numpy as jnp
from jax.experimental import pallas as pl
from jax.experimental.pallas import tpu as pltpu


R = 128
NBUF = 8


def _pe_add_kernel(scale_ref, in_hbm, pe_hbm, out_hbm,
                   in_vmem, pe_vmem, out_vmem,
                   in_sem, pe_sem, out_sem):
    nchunk = in_hbm.shape[1] // R
    s = scale_ref[0]

    def in_copy(j, slot):
        return pltpu.make_async_copy(
            in_hbm.at[:, pl.ds(j * R, R), :], in_vmem.at[slot], in_sem.at[slot])

    def pe_copy(j, slot):
        return pltpu.make_async_copy(
            pe_hbm.at[pl.ds(j * R, R), :], pe_vmem.at[slot], pe_sem.at[slot])

    def out_copy(j, slot):
        return pltpu.make_async_copy(
            out_vmem.at[slot], out_hbm.at[:, pl.ds(j * R, R), :], out_sem.at[slot])

    for k in range(NBUF):
        in_copy(k, k).start()
        pe_copy(k, k).start()

    def body(j, carry):
        slot = jax.lax.rem(j, NBUF)
        in_copy(j, slot).wait()
        pe_copy(j, slot).wait()

        @pl.when(j >= NBUF)
        def _():
            out_copy(j - NBUF, slot).wait()

        out_vmem[slot] = in_vmem[slot] + s * pe_vmem[slot][None, :, :]
        out_copy(j, slot).start()

        nxt = j + NBUF

        @pl.when(nxt < nchunk)
        def _():
            in_copy(nxt, slot).start()
            pe_copy(nxt, slot).start()

        return carry

    jax.lax.fori_loop(0, nchunk, body, 0)

    for k in range(NBUF):
        out_copy(nchunk - NBUF + k, k).wait()


def kernel(input, pe, scale_param):
    batch, seq, dim = input.shape
    return pl.pallas_call(
        _pe_add_kernel,
        in_specs=[
            pl.BlockSpec(memory_space=pltpu.SMEM),
            pl.BlockSpec(memory_space=pl.ANY),
            pl.BlockSpec(memory_space=pl.ANY),
        ],
        out_specs=pl.BlockSpec(memory_space=pl.ANY),
        out_shape=jax.ShapeDtypeStruct((batch, seq, dim), input.dtype),
        scratch_shapes=[
            pltpu.VMEM((NBUF, batch, R, dim), input.dtype),
            pltpu.VMEM((NBUF, R, dim), pe.dtype),
            pltpu.VMEM((NBUF, batch, R, dim), input.dtype),
            pltpu.SemaphoreType.DMA((NBUF,)),
            pltpu.SemaphoreType.DMA((NBUF,)),
            pltpu.SemaphoreType.DMA((NBUF,)),
        ],
    )(scale_param, input, pe[:seq])
